# baseline (speedup 1.0000x reference)
import functools

import jax
import jax.numpy as jnp
from jax import lax
from jax.experimental import pallas as pl
from jax.experimental.pallas import tpu as pltpu
from jax.experimental.pallas import tpu_sc as plsc


def _make_gather(B, V, D, b_per_w, num_cores):
    mesh = plsc.VectorSubcoreMesh(core_axis_name="c", subcore_axis_name="s")

    @functools.partial(
        pl.kernel,
        mesh=mesh,
        out_type=jax.ShapeDtypeStruct((B, D), jnp.float32),
        compiler_params=pltpu.CompilerParams(use_tc_tiling_on_sc=False),
        scratch_types=[
            pltpu.VMEM((b_per_w,), jnp.int32),
            pltpu.VMEM((b_per_w, D), jnp.float32),
            pltpu.SemaphoreType.DMA,
        ],
    )
    def gather_kernel(idx_hbm, table_hbm, out_hbm, idx_v, rows_v, sem):
        wid = lax.axis_index("s") * num_cores + lax.axis_index("c")
        base = wid * b_per_w
        pltpu.sync_copy(idx_hbm.at[pl.ds(base, b_per_w)], idx_v)
        pltpu.async_copy(table_hbm.at[idx_v], rows_v, sem).wait()
        pltpu.sync_copy(rows_v, out_hbm.at[pl.ds(base, b_per_w)])

    return gather_kernel


def kernel(context_ids, context_embeddings, prototypes):
    B = context_ids.shape[0]
    V, D = context_embeddings.shape
    info = plsc.get_sparse_core_info()
    nw = info.num_cores * info.num_subcores
    b_per_w = B // nw
    gather = _make_gather(B, V, D, b_per_w, info.num_cores)
    return gather(context_ids.astype(jnp.int32), context_embeddings)


# speedup vs baseline: 1.8995x; 1.8995x over previous
import functools

import jax
import jax.numpy as jnp
from jax import lax
from jax.experimental import pallas as pl
from jax.experimental.pallas import tpu as pltpu
from jax.experimental.pallas import tpu_sc as plsc

_OUT_CHUNK = 8192


def _make_rowgather(B, V, D, num_cores, num_subcores):
    nw = num_cores * num_subcores
    rows_per_w = D // nw
    mesh = plsc.VectorSubcoreMesh(core_axis_name="c", subcore_axis_name="s")
    n_chunks = B // _OUT_CHUNK

    @functools.partial(
        pl.kernel,
        mesh=mesh,
        out_type=jax.ShapeDtypeStruct((D, B), jnp.float32),
        compiler_params=pltpu.CompilerParams(needs_layout_passes=False),
        scratch_types=[
            pltpu.VMEM((V,), jnp.float32),
            pltpu.VMEM((B,), jnp.int32),
            pltpu.VMEM((_OUT_CHUNK,), jnp.float32),
        ],
    )
    def rowgather(idx_hbm, tableT_hbm, outT_hbm, row_v, idx_v, out_v):
        wid = lax.axis_index("s") * num_cores + lax.axis_index("c")
        pltpu.sync_copy(idx_hbm, idx_v)
        for r in range(rows_per_w):
            d = wid + r * nw
            pltpu.sync_copy(tableT_hbm.at[d], row_v)

            for chunk in range(n_chunks):
                cbase = chunk * _OUT_CHUNK

                def body(j, carry):
                    ids = idx_v[pl.ds(cbase + j * 16, 16)]
                    out_v[pl.ds(j * 16, 16)] = plsc.load_gather(row_v, [ids])
                    return carry

                lax.fori_loop(0, _OUT_CHUNK // 16, body, 0, unroll=8)
                pltpu.sync_copy(out_v, outT_hbm.at[d, pl.ds(cbase, _OUT_CHUNK)])

    return rowgather


def kernel(context_ids, context_embeddings, prototypes):
    B = context_ids.shape[0]
    V, D = context_embeddings.shape
    info = plsc.get_sparse_core_info()
    rowgather = _make_rowgather(B, V, D, info.num_cores, info.num_subcores)
    outT = rowgather(context_ids.astype(jnp.int32), context_embeddings.T)
    return outT.T


# speedup vs baseline: 2.7152x; 1.4294x over previous
import functools

import jax
import jax.numpy as jnp
from jax import lax
from jax.experimental import pallas as pl
from jax.experimental.pallas import tpu as pltpu
from jax.experimental.pallas import tpu_sc as plsc

_OUT_CHUNK = 4096


def _make_rowgather(B, V, D, num_cores, num_subcores):
    nw = num_cores * num_subcores
    rows_per_w = D // nw
    mesh = plsc.VectorSubcoreMesh(core_axis_name="c", subcore_axis_name="s")
    n_chunks = B // _OUT_CHUNK

    @functools.partial(
        pl.kernel,
        mesh=mesh,
        out_type=jax.ShapeDtypeStruct((D, B), jnp.float32),
        compiler_params=pltpu.CompilerParams(needs_layout_passes=False),
        scratch_types=[
            pltpu.VMEM((V,), jnp.float32),
            pltpu.VMEM((B,), jnp.int32),
            pltpu.VMEM((2, _OUT_CHUNK), jnp.float32),
            pltpu.SemaphoreType.DMA,
            pltpu.SemaphoreType.DMA,
        ],
    )
    def rowgather(idx_hbm, tableT_hbm, outT_hbm, row_v, idx_v, out_v, sem_in, sem_out):
        wid = lax.axis_index("s") * num_cores + lax.axis_index("c")
        idx_cp = pltpu.async_copy(idx_hbm, idx_v, sem_in)
        row_cp = pltpu.async_copy(tableT_hbm.at[wid], row_v, sem_in)
        idx_cp.wait()
        row_cp.wait()
        for r in range(rows_per_w):
            d = wid + r * nw
            out_cps = [None, None]
            for chunk in range(n_chunks):
                cbase = chunk * _OUT_CHUNK
                buf = chunk % 2
                if out_cps[buf] is not None:
                    out_cps[buf].wait()

                @plsc.parallel_loop(0, _OUT_CHUNK, step=16, unroll=8)
                def body(j, cbase=cbase, buf=buf):
                    ids = idx_v[pl.ds(cbase + j, 16)]
                    out_v[buf, pl.ds(j, 16)] = plsc.load_gather(row_v, [ids])
                out_cps[buf] = pltpu.async_copy(
                    out_v.at[buf], outT_hbm.at[d, pl.ds(cbase, _OUT_CHUNK)], sem_out
                )
            for cp in out_cps:
                cp.wait()
            if r + 1 < rows_per_w:
                pltpu.sync_copy(tableT_hbm.at[d + nw], row_v)

    return rowgather


def kernel(context_ids, context_embeddings, prototypes):
    B = context_ids.shape[0]
    V, D = context_embeddings.shape
    info = plsc.get_sparse_core_info()
    rowgather = _make_rowgather(B, V, D, info.num_cores, info.num_subcores)
    outT = rowgather(context_ids.astype(jnp.int32), context_embeddings.T)
    return outT.T
